# Initial kernel scaffold; baseline (speedup 1.0000x reference)
import jax
import jax.numpy as jnp
from jax.experimental import pallas as pl


def kernel(x1, x2, mask, x1_lora_a, x2_lora_a, w1_lora_b, w2_lora_b, batch_size):
    raise NotImplementedError("write your pallas kernel here")


# speedup vs baseline: 21.2702x; 21.2702x over previous
import functools

import jax
import jax.numpy as jnp
import numpy as np
from jax import lax
from jax.experimental import pallas as pl
from jax.experimental.pallas import tpu as pltpu
from jax.experimental.pallas import tpu_sc as plsc

NC, NS, L = 2, 16, 16
NW = NC * NS
BINS = 4096
BH, BW = 32, 128
I32MIN = np.int32(-2147483648)


def _prod_body(x1_ref, x2_ref, a2_ref, w2_ref, res_ref, x2m_ref):
  x1 = x1_ref[...]
  x2 = x2_ref[...]
  res_ref[...] = x1 * x2
  r2 = jnp.dot(a2_ref[0], w2_ref[...], preferred_element_type=jnp.float32)
  x2m_ref[...] = lax.bitcast_convert_type(x2 - r2[None], jnp.int32)


def _product_and_x2main(x1, x2, a2, w2, sb):
  b, s, d = x1.shape
  r = w2.shape[0]
  grid = (b, s // sb)
  return pl.pallas_call(
      _prod_body,
      grid=grid,
      in_specs=[
          pl.BlockSpec((1, sb, d), lambda i, j: (i, j, 0)),
          pl.BlockSpec((1, sb, d), lambda i, j: (i, j, 0)),
          pl.BlockSpec((1, sb, r), lambda i, j: (i, j, 0)),
          pl.BlockSpec((r, d), lambda i, j: (0, 0)),
      ],
      out_specs=[
          pl.BlockSpec((1, sb, d), lambda i, j: (i, j, 0)),
          pl.BlockSpec((1, sb, d), lambda i, j: (i, j, 0)),
      ],
      out_shape=[
          jax.ShapeDtypeStruct((b, s, d), jnp.float32),
          jax.ShapeDtypeStruct((b, s, d), jnp.int32),
      ],
  )(x1, x2, a2, w2)


def _sc_hist_pass(n, ch, shift, hs, use_mask, interpret=False):
  per_tile = n // NW
  n_chunks = per_tile // ch
  rows = L * BH
  mesh = plsc.VectorSubcoreMesh(
      core_axis_name="c", subcore_axis_name="s", num_cores=NC,
      num_subcores=NS)

  def body(d1_hbm, d2_hbm, pfx_hbm, out_hbm, buf, hist, pfxv):
    wid = lax.axis_index("s") * NC + lax.axis_index("c")
    lane_bins = lax.iota(jnp.int32, 16) * BINS
    zeros = jnp.zeros((16,), jnp.int32)
    ones = jnp.ones((16,), jnp.int32)

    for j in range(2):
      d_hbm = (d1_hbm, d2_hbm)[j]

      def zloop(i, carry):
        hist[pl.ds(i * 16, 16)] = zeros
        return carry
      lax.fori_loop(0, (L * BINS) // 16, zloop, 0)

      pltpu.sync_copy(pfx_hbm.at[j], pfxv)
      pfx = pfxv[...]

      base = wid * per_tile

      def chunk_body(c, carry):
        pltpu.sync_copy(d_hbm.at[pl.ds(base + c * ch, ch)], buf)

        def vec_body(i, carry2):
          bits = buf[pl.ds(i * 16, 16)]
          m = lax.shift_right_arithmetic(bits, 31) | I32MIN
          key = bits ^ m
          digit = lax.shift_right_logical(key, shift) & (BINS - 1)
          idx = lane_bins + digit
          if use_mask:
            match = lax.shift_right_logical(key, hs) == pfx
            plsc.addupdate_scatter(hist, [idx], ones, mask=match)
          else:
            plsc.addupdate_scatter(hist, [idx], ones)
          return carry2

        lax.fori_loop(0, ch // 16, vec_body, 0)
        return carry

      lax.fori_loop(0, n_chunks, chunk_body, 0)

      pltpu.sync_copy(hist, out_hbm.at[j, wid])

  return pl.kernel(
      body,
      out_type=jax.ShapeDtypeStruct((2, NW, L * BINS), jnp.int32),
      mesh=mesh,
      scratch_types=[
          pltpu.VMEM((ch,), jnp.int32),
          pltpu.VMEM((L * BINS,), jnp.int32),
          pltpu.VMEM((16,), jnp.int32),
      ],
      compiler_params=pltpu.CompilerParams(needs_layout_passes=False),
      interpret=interpret,
  )


def _find_body(shift, h_ref, kr_ref, pfx_ref, pfxf_ref, pfxs_ref, kro_ref,
               vals_ref):
  h = h_ref[...].astype(jnp.float32)
  h = h.reshape(2, NW * L, BH, BW)
  cnt = jnp.sum(h, axis=1)

  row = lax.broadcasted_iota(jnp.int32, (BW, BW), 0)
  col = lax.broadcasted_iota(jnp.int32, (BW, BW), 1)
  u128 = (row <= col).astype(jnp.float32)
  r2 = lax.broadcasted_iota(jnp.int32, (BH, BH), 0)
  c2 = lax.broadcasted_iota(jnp.int32, (BH, BH), 1)
  s32 = (c2 < r2).astype(jnp.float32)

  lin = (lax.broadcasted_iota(jnp.int32, (BH, BW), 0) * BW
         + lax.broadcasted_iota(jnp.int32, (BH, BW), 1))
  big = np.int32(1 << 30)

  for j in range(2):
    hj = cnt[j]
    p = jnp.dot(hj, u128, preferred_element_type=jnp.float32)
    rowsum = p[:, BW - 1:BW]
    e = jnp.dot(s32, rowsum, preferred_element_type=jnp.float32)
    c = p + e

    krj = kr_ref[j:j + 1, 0:1].astype(jnp.float32)
    krb = jnp.broadcast_to(krj, (BH, BW))
    cand = jnp.where(c >= krb, lin, big)
    dstar = jnp.min(cand, axis=(0, 1), keepdims=True)
    dstar_b = jnp.broadcast_to(dstar, (BH, BW))
    sel = lin == dstar_b
    c_at = jnp.sum(jnp.where(sel, c, 0.0), axis=(0, 1), keepdims=True)
    h_at = jnp.sum(jnp.where(sel, hj, 0.0), axis=(0, 1), keepdims=True)
    below = (c_at - h_at).astype(jnp.int32)

    kr_in = kr_ref[j:j + 1, :]
    kro_ref[j:j + 1, :] = kr_in - jnp.broadcast_to(below, (1, 16))

    pfx_in = pfx_ref[j:j + 1, :]
    d16 = jnp.broadcast_to(dstar, (1, 16))
    pfx_full = pfx_in | lax.shift_left(d16, shift)
    pfxf_ref[j:j + 1, :] = pfx_full
    pfxs_ref[j:j + 1, :] = lax.shift_right_logical(pfx_full, shift)

    s2 = lax.shift_right_logical(pfx_full, 31)
    m2 = (s2 - 1) | I32MIN
    vals_ref[j:j + 1, :] = lax.bitcast_convert_type(pfx_full ^ m2,
                                                    jnp.float32)


def _find_bin(shift, interpret=False):
  return pl.pallas_call(
      functools.partial(_find_body, shift),
      out_shape=[
          jax.ShapeDtypeStruct((2, 16), jnp.int32),
          jax.ShapeDtypeStruct((2, 16), jnp.int32),
          jax.ShapeDtypeStruct((2, 16), jnp.int32),
          jax.ShapeDtypeStruct((2, 16), jnp.float32),
      ],
      interpret=interpret,
  )


def _kth_values(x1f, x2mf, k1, k2, ch, interpret=False):
  n = x1f.shape[0]
  kr0 = jnp.asarray([[k1] * 16, [k2] * 16], dtype=jnp.int32)
  pfx0 = jnp.zeros((2, 16), jnp.int32)

  h1 = _sc_hist_pass(n, ch, 20, 32, False, interpret)(x1f, x2mf, pfx0)
  pfxf1, pfxs1, kr1, _ = _find_bin(20, interpret)(
      h1.reshape(2, NW, L * BH, BW), kr0, pfx0)

  h2 = _sc_hist_pass(n, ch, 8, 20, True, interpret)(x1f, x2mf, pfxs1)
  pfxf2, pfxs2, kr2, _ = _find_bin(8, interpret)(
      h2.reshape(2, NW, L * BH, BW), kr1, pfxf1)

  h3 = _sc_hist_pass(n, ch, 0, 8, True, interpret)(x1f, x2mf, pfxs2)
  _, _, _, vals = _find_bin(0, interpret)(
      h3.reshape(2, NW, L * BH, BW), kr2, pfxf2)
  return vals[0, 0], vals[1, 0]


def kernel(x1, x2, mask, x1_lora_a, x2_lora_a, w1_lora_b, w2_lora_b,
           batch_size):
  del mask, x1_lora_a, w1_lora_b, batch_size
  n = x1.size
  k1 = int(n * 0.9)
  k2 = k1
  result, x2m = _product_and_x2main(x1, x2, x2_lora_a, w2_lora_b, sb=256)
  x1f = lax.bitcast_convert_type(x1, jnp.int32).reshape(n)
  x2mf = x2m.reshape(n)
  kth1, kth2 = _kth_values(x1f, x2mf, k1, k2, ch=16384)
  return result, kth1, kth2
